# baseline (speedup 1.0000x reference)
import functools

import jax
import jax.numpy as jnp
from jax import lax
from jax.experimental import pallas as pl
from jax.experimental.pallas import tpu as pltpu
from jax.experimental.pallas import tpu_sc as plsc

BATCH = 16384
NSPLIT = 2
B_HALF = BATCH // NSPLIT
EMB_D = 32
DOW_D = 7
HOD_D = 24
OH_D = DOW_D + HOD_D
OUT_D = EMB_D + OH_D
L = 16
NC, NS = 2, 16
NW = NC * NS
B_PER_W = B_HALF // NW
GCHUNK = 128
NCH = B_PER_W // GCHUNK
PAD_D = 128


def _sc_body(base_rows, uid_hbm, dow_hbm, hod_hbm, tab_hbm, out_hbm,
             uid_v, dow_v, hod_v, idx_v, ebuf, buf, gsem, osem):
    wid = lax.axis_index("s") * NC + lax.axis_index("c")
    src_base = base_rows + wid * B_PER_W
    base = wid * B_PER_W
    vocab = tab_hbm.shape[0] - 1

    pltpu.sync_copy(uid_hbm.at[pl.ds(src_base, B_PER_W)], uid_v)
    pltpu.sync_copy(dow_hbm.at[pl.ds(src_base, B_PER_W)], dow_v)
    pltpu.sync_copy(hod_hbm.at[pl.ds(src_base, B_PER_W)], hod_v)

    gcopies = []
    for j in range(NCH):
        @plsc.parallel_loop(j * (GCHUNK // L), (j + 1) * (GCHUNK // L),
                            unroll=4)
        def _(i):
            u = uid_v[pl.ds(i * L, L)]
            ok = (u >= 0) & (u < vocab)
            idx_v[pl.ds(i * L, L)] = jnp.where(ok, u + 1, 0)

        gcopies.append(pltpu.async_copy(
            tab_hbm.at[idx_v.at[pl.ds(j * GCHUNK, GCHUNK)]],
            ebuf.at[pl.ds(j * GCHUNK, GCHUNK)], gsem.at[j]))

    zeros = jnp.zeros((L,), jnp.float32)
    ones = jnp.full((L,), 1.0, jnp.float32)
    rows0 = lax.iota(jnp.int32, L)
    ocopies = []
    for j in range(NCH):
        gcopies[j].wait()

        @plsc.parallel_loop(j * GCHUNK, (j + 1) * GCHUNK, unroll=4)
        def _(r):
            buf[r, pl.ds(0, L)] = ebuf[r, pl.ds(0, L)]
            buf[r, pl.ds(L, L)] = ebuf[r, pl.ds(L, L)]
            buf[r, pl.ds(EMB_D, L)] = zeros
            buf[r, pl.ds(3 * L, L)] = zeros

        @plsc.parallel_loop(j * (GCHUNK // L), (j + 1) * (GCHUNK // L),
                            unroll=2)
        def _(i):
            rows = rows0 + i * L
            d = dow_v[pl.ds(i * L, L)]
            h = hod_v[pl.ds(i * L, L)]
            plsc.store_scatter(buf, [rows, d + EMB_D], ones)
            plsc.store_scatter(buf, [rows, h + (EMB_D + DOW_D)], ones)

        ocopies.append(pltpu.async_copy(
            buf.at[pl.ds(j * GCHUNK, GCHUNK)],
            out_hbm.at[pl.ds(base + j * GCHUNK, GCHUNK), pl.ds(0, 2 * EMB_D)],
            osem))
    for cp in ocopies:
        cp.wait()


def _make_half(base_rows):
    @functools.partial(
        pl.kernel,
        out_type=jax.ShapeDtypeStruct((B_HALF, PAD_D), jnp.float32),
        mesh=plsc.VectorSubcoreMesh(core_axis_name="c", subcore_axis_name="s",
                                    num_cores=NC, num_subcores=NS),
        scratch_types=[
            pltpu.VMEM((B_PER_W,), jnp.int32),
            pltpu.VMEM((B_PER_W,), jnp.int32),
            pltpu.VMEM((B_PER_W,), jnp.int32),
            pltpu.VMEM((B_PER_W,), jnp.int32),
            pltpu.VMEM((B_PER_W, EMB_D), jnp.float32),
            pltpu.VMEM((B_PER_W, 2 * EMB_D), jnp.float32),
            pltpu.SemaphoreType.DMA((NCH,)),
            pltpu.SemaphoreType.DMA,
        ],
        compiler_params=pltpu.CompilerParams(use_tc_tiling_on_sc=False,
                                             needs_layout_passes=False),
    )
    def _half(uid_hbm, dow_hbm, hod_hbm, tab_hbm, out_hbm,
              uid_v, dow_v, hod_v, idx_v, ebuf, buf, gsem, osem):
        _sc_body(base_rows, uid_hbm, dow_hbm, hod_hbm, tab_hbm, out_hbm,
                 uid_v, dow_v, hod_v, idx_v, ebuf, buf, gsem, osem)

    return _half


_sc_half0 = _make_half(0)
_sc_half1 = _make_half(B_HALF)


def kernel(user_id, dow, hod, table):
    p0 = _sc_half0(user_id, dow, hod, table)
    p1 = _sc_half1(user_id, dow, hod, table)
    return jnp.concatenate(
        [lax.slice(p0, (0, 0), (B_HALF, OUT_D)),
         lax.slice(p1, (0, 0), (B_HALF, OUT_D))], axis=0)


# speedup vs baseline: 1.3262x; 1.3262x over previous
import functools

import jax
import jax.numpy as jnp
from jax import lax
from jax.experimental import pallas as pl
from jax.experimental.pallas import tpu as pltpu
from jax.experimental.pallas import tpu_sc as plsc

BATCH = 16384
EMB_D = 32
DOW_D = 7
HOD_D = 24
OH_D = DOW_D + HOD_D
OUT_D = EMB_D + OH_D
L = 16
NC, NS = 2, 16
NW = NC * NS
B_PER_W = BATCH // NW
GCHUNK = 128
NCH = B_PER_W // GCHUNK
PAD_D = 128


def _sc_body(uid_hbm, dow_hbm, hod_hbm, tab_hbm, out_hbm,
             uid_v, dow_v, hod_v, idx_v, ebuf, buf, gsem, osem):
    wid = lax.axis_index("s") * NC + lax.axis_index("c")
    base = wid * B_PER_W
    vocab = tab_hbm.shape[0] - 1

    pltpu.sync_copy(uid_hbm.at[pl.ds(base, B_PER_W)], uid_v)
    pltpu.sync_copy(dow_hbm.at[pl.ds(base, B_PER_W)], dow_v)
    pltpu.sync_copy(hod_hbm.at[pl.ds(base, B_PER_W)], hod_v)

    gcopies = []
    for j in range(NCH):
        @plsc.parallel_loop(j * (GCHUNK // L), (j + 1) * (GCHUNK // L),
                            unroll=4)
        def _(i):
            u = uid_v[pl.ds(i * L, L)]
            ok = (u >= 0) & (u < vocab)
            idx_v[pl.ds(i * L, L)] = jnp.where(ok, u + 1, 0)

        gcopies.append(pltpu.async_copy(
            tab_hbm.at[idx_v.at[pl.ds(j * GCHUNK, GCHUNK)]],
            ebuf.at[pl.ds(j * GCHUNK, GCHUNK)], gsem.at[j]))

    zeros = jnp.zeros((L,), jnp.float32)
    ones = jnp.full((L,), 1.0, jnp.float32)
    rows0 = lax.iota(jnp.int32, L)
    ocopies = []
    for j in range(NCH):
        gcopies[j].wait()

        @plsc.parallel_loop(j * GCHUNK, (j + 1) * GCHUNK, unroll=8)
        def _(r):
            buf[r, pl.ds(0, L)] = zeros
            buf[r, pl.ds(L, L)] = zeros

        @plsc.parallel_loop(j * (GCHUNK // L), (j + 1) * (GCHUNK // L),
                            unroll=2)
        def _(i):
            rows = rows0 + i * L
            d = dow_v[pl.ds(i * L, L)]
            h = hod_v[pl.ds(i * L, L)]
            plsc.store_scatter(buf, [rows, d], ones)
            plsc.store_scatter(buf, [rows, h + DOW_D], ones)

        ocopies.append(pltpu.async_copy(
            ebuf.at[pl.ds(j * GCHUNK, GCHUNK)],
            out_hbm.at[pl.ds(base + j * GCHUNK, GCHUNK), pl.ds(0, EMB_D)],
            osem))
        ocopies.append(pltpu.async_copy(
            buf.at[pl.ds(j * GCHUNK, GCHUNK)],
            out_hbm.at[pl.ds(base + j * GCHUNK, GCHUNK), pl.ds(EMB_D, EMB_D)],
            osem))
    for cp in ocopies:
        cp.wait()


@functools.partial(
    pl.kernel,
    out_type=jax.ShapeDtypeStruct((BATCH, PAD_D), jnp.float32),
    mesh=plsc.VectorSubcoreMesh(core_axis_name="c", subcore_axis_name="s",
                                num_cores=NC, num_subcores=NS),
    scratch_types=[
        pltpu.VMEM((B_PER_W,), jnp.int32),
        pltpu.VMEM((B_PER_W,), jnp.int32),
        pltpu.VMEM((B_PER_W,), jnp.int32),
        pltpu.VMEM((B_PER_W,), jnp.int32),
        pltpu.VMEM((B_PER_W, EMB_D), jnp.float32),
        pltpu.VMEM((B_PER_W, EMB_D), jnp.float32),
        pltpu.SemaphoreType.DMA((NCH,)),
        pltpu.SemaphoreType.DMA,
    ],
    compiler_params=pltpu.CompilerParams(use_tc_tiling_on_sc=False,
                                         needs_layout_passes=False),
)
def _sc_query_model(uid_hbm, dow_hbm, hod_hbm, tab_hbm, out_hbm,
                    uid_v, dow_v, hod_v, idx_v, ebuf, buf, gsem, osem):
    _sc_body(uid_hbm, dow_hbm, hod_hbm, tab_hbm, out_hbm,
             uid_v, dow_v, hod_v, idx_v, ebuf, buf, gsem, osem)


def kernel(user_id, dow, hod, table):
    padded = _sc_query_model(user_id, dow, hod, table)
    return lax.slice(padded, (0, 0), (BATCH, OUT_D))


# speedup vs baseline: 1.3548x; 1.0216x over previous
import functools

import jax
import jax.numpy as jnp
from jax import lax
from jax.experimental import pallas as pl
from jax.experimental.pallas import tpu as pltpu
from jax.experimental.pallas import tpu_sc as plsc

BATCH = 16384
EMB_D = 32
DOW_D = 7
HOD_D = 24
OH_D = DOW_D + HOD_D
OUT_D = EMB_D + OH_D
L = 16
NC, NS = 2, 16
NW = NC * NS
B_PER_W = BATCH // NW
GCHUNK = 128
NCH = B_PER_W // GCHUNK
PAD_D = 128


def _sc_body(uid_hbm, dow_hbm, hod_hbm, tab_hbm, out_hbm,
             uid_v, dow_v, hod_v, idx_v, ebuf, buf, gsem, osem):
    wid = lax.axis_index("s") * NC + lax.axis_index("c")
    base = wid * B_PER_W
    vocab = tab_hbm.shape[0] - 1

    pltpu.sync_copy(uid_hbm.at[pl.ds(base, B_PER_W)], uid_v)
    pltpu.sync_copy(dow_hbm.at[pl.ds(base, B_PER_W)], dow_v)
    pltpu.sync_copy(hod_hbm.at[pl.ds(base, B_PER_W)], hod_v)

    gcopies = []
    for j in range(NCH):
        @plsc.parallel_loop(j * (GCHUNK // L), (j + 1) * (GCHUNK // L),
                            unroll=4)
        def _(i):
            u = uid_v[pl.ds(i * L, L)]
            ok = (u >= 0) & (u < vocab)
            idx_v[pl.ds(i * L, L)] = jnp.where(ok, u + 1, 0)

        gcopies.append(pltpu.async_copy(
            tab_hbm.at[idx_v.at[pl.ds(j * GCHUNK, GCHUNK)]],
            ebuf.at[pl.ds(j * GCHUNK, GCHUNK)], gsem.at[j]))

    zeros = jnp.zeros((L,), jnp.float32)
    ones = jnp.full((L,), 1.0, jnp.float32)
    rows0 = lax.iota(jnp.int32, L)

    @plsc.parallel_loop(0, B_PER_W, unroll=8)
    def _(r):
        buf[r, pl.ds(0, L)] = zeros
        buf[r, pl.ds(L, L)] = zeros

    @plsc.parallel_loop(0, B_PER_W // L, unroll=2)
    def _(i):
        rows = rows0 + i * L
        d = dow_v[pl.ds(i * L, L)]
        h = hod_v[pl.ds(i * L, L)]
        plsc.store_scatter(buf, [rows, d], ones)
        plsc.store_scatter(buf, [rows, h + DOW_D], ones)

    ocopies = [pltpu.async_copy(
        buf, out_hbm.at[pl.ds(base, B_PER_W), pl.ds(EMB_D, EMB_D)], osem)]
    for j in range(NCH):
        gcopies[j].wait()
        ocopies.append(pltpu.async_copy(
            ebuf.at[pl.ds(j * GCHUNK, GCHUNK)],
            out_hbm.at[pl.ds(base + j * GCHUNK, GCHUNK), pl.ds(0, EMB_D)],
            osem))
    for cp in ocopies:
        cp.wait()


@functools.partial(
    pl.kernel,
    out_type=jax.ShapeDtypeStruct((BATCH, PAD_D), jnp.float32),
    mesh=plsc.VectorSubcoreMesh(core_axis_name="c", subcore_axis_name="s",
                                num_cores=NC, num_subcores=NS),
    scratch_types=[
        pltpu.VMEM((B_PER_W,), jnp.int32),
        pltpu.VMEM((B_PER_W,), jnp.int32),
        pltpu.VMEM((B_PER_W,), jnp.int32),
        pltpu.VMEM((B_PER_W,), jnp.int32),
        pltpu.VMEM((B_PER_W, EMB_D), jnp.float32),
        pltpu.VMEM((B_PER_W, EMB_D), jnp.float32),
        pltpu.SemaphoreType.DMA((NCH,)),
        pltpu.SemaphoreType.DMA,
    ],
    compiler_params=pltpu.CompilerParams(use_tc_tiling_on_sc=False,
                                         needs_layout_passes=False),
)
def _sc_query_model(uid_hbm, dow_hbm, hod_hbm, tab_hbm, out_hbm,
                    uid_v, dow_v, hod_v, idx_v, ebuf, buf, gsem, osem):
    _sc_body(uid_hbm, dow_hbm, hod_hbm, tab_hbm, out_hbm,
             uid_v, dow_v, hod_v, idx_v, ebuf, buf, gsem, osem)


def kernel(user_id, dow, hod, table):
    padded = _sc_query_model(user_id, dow, hod, table)
    return lax.slice(padded, (0, 0), (BATCH, OUT_D))


# speedup vs baseline: 1.3958x; 1.0302x over previous
import functools

import jax
import jax.numpy as jnp
from jax import lax
from jax.experimental import pallas as pl
from jax.experimental.pallas import tpu as pltpu
from jax.experimental.pallas import tpu_sc as plsc

BATCH = 16384
EMB_D = 32
DOW_D = 7
HOD_D = 24
OH_D = DOW_D + HOD_D
OUT_D = EMB_D + OH_D
L = 16
NC, NS = 2, 16
NW = NC * NS
B_PER_W = BATCH // NW
GCHUNK = 128
NCH = B_PER_W // GCHUNK
PAD_D = 128


def _sc_body(uid_hbm, dow_hbm, hod_hbm, tab_hbm, out_hbm,
             uid_v, dow_v, hod_v, idx_v, ebuf, buf, gsem, osem, ssem):
    wid = lax.axis_index("s") * NC + lax.axis_index("c")
    base = wid * B_PER_W
    vocab = tab_hbm.shape[0] - 1

    cp_u = pltpu.async_copy(uid_hbm.at[pl.ds(base, B_PER_W)], uid_v, ssem.at[0])
    cp_d = pltpu.async_copy(dow_hbm.at[pl.ds(base, B_PER_W)], dow_v, ssem.at[1])
    cp_h = pltpu.async_copy(hod_hbm.at[pl.ds(base, B_PER_W)], hod_v, ssem.at[2])
    cp_u.wait()

    gcopies = []
    for j in range(NCH):
        @plsc.parallel_loop(j * (GCHUNK // L), (j + 1) * (GCHUNK // L),
                            unroll=4)
        def _(i):
            u = uid_v[pl.ds(i * L, L)]
            ok = (u >= 0) & (u < vocab)
            idx_v[pl.ds(i * L, L)] = jnp.where(ok, u + 1, 0)

        gcopies.append(pltpu.async_copy(
            tab_hbm.at[idx_v.at[pl.ds(j * GCHUNK, GCHUNK)]],
            ebuf.at[pl.ds(j * GCHUNK, GCHUNK)], gsem.at[j]))

    zeros = jnp.zeros((L,), jnp.float32)
    ones = jnp.full((L,), 1.0, jnp.float32)
    rows0 = lax.iota(jnp.int32, L)

    @plsc.parallel_loop(0, B_PER_W, unroll=8)
    def _(r):
        buf[r, pl.ds(0, L)] = zeros
        buf[r, pl.ds(L, L)] = zeros

    cp_d.wait()
    cp_h.wait()

    @plsc.parallel_loop(0, B_PER_W // L, unroll=2)
    def _(i):
        rows = rows0 + i * L
        d = dow_v[pl.ds(i * L, L)]
        h = hod_v[pl.ds(i * L, L)]
        plsc.store_scatter(buf, [rows, d], ones)
        plsc.store_scatter(buf, [rows, h + DOW_D], ones)

    ocopies = [pltpu.async_copy(
        buf, out_hbm.at[pl.ds(base, B_PER_W), pl.ds(EMB_D, EMB_D)], osem)]
    for j in range(NCH):
        gcopies[j].wait()
        ocopies.append(pltpu.async_copy(
            ebuf.at[pl.ds(j * GCHUNK, GCHUNK)],
            out_hbm.at[pl.ds(base + j * GCHUNK, GCHUNK), pl.ds(0, EMB_D)],
            osem))
    for cp in ocopies:
        cp.wait()


@functools.partial(
    pl.kernel,
    out_type=jax.ShapeDtypeStruct((BATCH, PAD_D), jnp.float32),
    mesh=plsc.VectorSubcoreMesh(core_axis_name="c", subcore_axis_name="s",
                                num_cores=NC, num_subcores=NS),
    scratch_types=[
        pltpu.VMEM((B_PER_W,), jnp.int32),
        pltpu.VMEM((B_PER_W,), jnp.int32),
        pltpu.VMEM((B_PER_W,), jnp.int32),
        pltpu.VMEM((B_PER_W,), jnp.int32),
        pltpu.VMEM((B_PER_W, EMB_D), jnp.float32),
        pltpu.VMEM((B_PER_W, EMB_D), jnp.float32),
        pltpu.SemaphoreType.DMA((NCH,)),
        pltpu.SemaphoreType.DMA,
        pltpu.SemaphoreType.DMA((3,)),
    ],
    compiler_params=pltpu.CompilerParams(use_tc_tiling_on_sc=False,
                                         needs_layout_passes=False),
)
def _sc_query_model(uid_hbm, dow_hbm, hod_hbm, tab_hbm, out_hbm,
                    uid_v, dow_v, hod_v, idx_v, ebuf, buf, gsem, osem, ssem):
    _sc_body(uid_hbm, dow_hbm, hod_hbm, tab_hbm, out_hbm,
             uid_v, dow_v, hod_v, idx_v, ebuf, buf, gsem, osem, ssem)


def kernel(user_id, dow, hod, table):
    padded = _sc_query_model(user_id, dow, hod, table)
    return lax.slice(padded, (0, 0), (BATCH, OUT_D))
